# Initial kernel scaffold; baseline (speedup 1.0000x reference)
import jax
import jax.numpy as jnp
from jax.experimental import pallas as pl


def kernel(x, edge_index, W1, b1, W2, b2):
    raise NotImplementedError("write your pallas kernel here")


# speedup vs baseline: 15.1051x; 15.1051x over previous
import functools

import jax
import jax.numpy as jnp
from jax import lax
from jax.experimental import pallas as pl
from jax.experimental.pallas import tpu as pltpu
from jax.experimental.pallas import tpu_sc as plsc

N = 10000
E = 320000
D_IN = 128
D = 128
D_OUT = 40
D_OUT_PAD = 48

NC = 2
NS = 16
NT = NC * NS
CHUNK = 128
BASE_CHUNKS = 78
EXTRA_BASE = NT * BASE_CHUNKS * CHUNK
EXTRA_TILES = (E - EXTRA_BASE) // CHUNK

ROWS_A = 624
TAIL0 = NS * ROWS_A
TAIL = N - TAIL0
Z_LENS = (128, 128, 128, 128, 112)


def _sc_mesh():
    return plsc.VectorSubcoreMesh(
        core_axis_name="c", subcore_axis_name="s", num_cores=NC, num_subcores=NS
    )


def _fill_rows(rows_ref, nrows, value):
    vec = jnp.full((16,), value, jnp.float32)

    @pl.loop(0, nrows)
    def _(r):
        @pl.loop(0, D // 16)
        def _(l):
            rows_ref.at[r][pl.ds(l * 16, 16)] = vec


def _zero_acc_slice(rows, acc, sid):
    row0 = sid * ROWS_A
    off = 0
    for ln in Z_LENS:
        pltpu.sync_copy(rows.at[pl.ds(0, ln)], acc.at[pl.ds(row0 + off, ln)])
        off += ln

    @pl.when(sid == 0)
    def _():
        pltpu.sync_copy(rows.at[pl.ds(0, TAIL)], acc.at[pl.ds(TAIL0, TAIL)])


def _write_back(acc, out_hbm, cid, sid):
    row0 = sid * ROWS_A
    pltpu.sync_copy(acc.at[pl.ds(row0, ROWS_A)],
                    out_hbm.at[cid, pl.ds(row0, ROWS_A)])

    @pl.when(sid == 0)
    def _():
        pltpu.sync_copy(acc.at[pl.ds(TAIL0, TAIL)],
                        out_hbm.at[cid, pl.ds(TAIL0, TAIL)])


@functools.cache
def _make_deg_kernel():
    @functools.partial(
        pl.kernel,
        out_type=jax.ShapeDtypeStruct((NC, N, D), jnp.float32),
        mesh=_sc_mesh(),
        scratch_types=[
            pltpu.VMEM((CHUNK,), jnp.int32),
            pltpu.VMEM((CHUNK, D), jnp.float32),
            pltpu.VMEM_SHARED((N, D), jnp.float32),
        ],
    )
    def deg_kernel(dst_hbm, out_hbm, idx_d, rows, acc):
        cid = lax.axis_index("c")
        sid = lax.axis_index("s")
        tid = cid * NS + sid

        _fill_rows(rows, CHUNK, 0.0)
        _zero_acc_slice(rows, acc, sid)
        plsc.subcore_barrier()

        _fill_rows(rows, CHUNK, 1.0)
        e0 = tid * (BASE_CHUNKS * CHUNK)

        @pl.loop(0, BASE_CHUNKS)
        def _(i):
            pltpu.sync_copy(dst_hbm.at[pl.ds(e0 + i * CHUNK, CHUNK)], idx_d)
            pltpu.sync_copy(rows, acc.at[idx_d], add=True)

        @pl.when(tid < EXTRA_TILES)
        def _():
            pltpu.sync_copy(dst_hbm.at[pl.ds(EXTRA_BASE + tid * CHUNK, CHUNK)],
                            idx_d)
            pltpu.sync_copy(rows, acc.at[idx_d], add=True)

        plsc.subcore_barrier()
        _write_back(acc, out_hbm, cid, sid)

    return deg_kernel


@functools.cache
def _make_agg_kernel():
    @functools.partial(
        pl.kernel,
        out_type=jax.ShapeDtypeStruct((NC, N, D), jnp.float32),
        mesh=_sc_mesh(),
        scratch_types=[
            pltpu.VMEM((CHUNK,), jnp.int32),
            pltpu.VMEM((CHUNK,), jnp.int32),
            pltpu.VMEM((CHUNK, D), jnp.float32),
            pltpu.VMEM_SHARED((N, D), jnp.float32),
            pltpu.SemaphoreType.DMA,
        ],
    )
    def agg_kernel(g_hbm, src_hbm, dst_hbm, out_hbm, idx_s, idx_d, rows, acc,
                   sem):
        cid = lax.axis_index("c")
        sid = lax.axis_index("s")
        tid = cid * NS + sid

        _fill_rows(rows, CHUNK, 0.0)
        _zero_acc_slice(rows, acc, sid)
        plsc.subcore_barrier()

        e0 = tid * (BASE_CHUNKS * CHUNK)

        def do_chunk(base):
            pltpu.sync_copy(src_hbm.at[pl.ds(base, CHUNK)], idx_s)
            pltpu.sync_copy(dst_hbm.at[pl.ds(base, CHUNK)], idx_d)
            pltpu.async_copy(g_hbm.at[idx_s], rows, sem).wait()
            pltpu.sync_copy(rows, acc.at[idx_d], add=True)

        @pl.loop(0, BASE_CHUNKS)
        def _(i):
            do_chunk(e0 + i * CHUNK)

        @pl.when(tid < EXTRA_TILES)
        def _():
            do_chunk(EXTRA_BASE + tid * CHUNK)

        plsc.subcore_barrier()
        _write_back(acc, out_hbm, cid, sid)

    return agg_kernel


def _m1_body(x_ref, w_ref, degp_ref, g1_ref, dinv_ref):
    deg = degp_ref[0, :, 0:1] + degp_ref[1, :, 0:1] + 1.0
    dinv = lax.rsqrt(deg)
    dinv_ref[...] = dinv
    h = jnp.dot(x_ref[...], w_ref[...], preferred_element_type=jnp.float32)
    g1_ref[...] = h * dinv


def _m2_body(p_ref, g1_ref, dinv_ref, b1_ref, u_ref):
    s = p_ref[0] + p_ref[1] + g1_ref[...]
    dinv = dinv_ref[...]
    h1 = jnp.maximum(s * dinv + b1_ref[...], 0.0)
    u_ref[...] = h1 * dinv


def _e3_body(p_ref, u_ref, dinv_ref, b2_ref, w2_ref, out_ref):
    s = p_ref[0] + p_ref[1] + u_ref[...]
    h2 = jnp.dot(s, w2_ref[...], preferred_element_type=jnp.float32)
    out_ref[...] = h2 * dinv_ref[...] + b2_ref[...]


_m1 = pl.pallas_call(
    _m1_body,
    out_shape=(jax.ShapeDtypeStruct((N, D), jnp.float32),
               jax.ShapeDtypeStruct((N, 1), jnp.float32)),
)
_m2 = pl.pallas_call(
    _m2_body,
    out_shape=jax.ShapeDtypeStruct((N, D), jnp.float32),
)
_e3 = pl.pallas_call(
    _e3_body,
    out_shape=jax.ShapeDtypeStruct((N, D_OUT_PAD), jnp.float32),
)


@jax.jit
def _run(x, edge_index, W1, b1, W2, b2):
    ei = edge_index.astype(jnp.int32)
    src = ei[0]
    dst = ei[1]
    w2p = jnp.pad(W2, ((0, 0), (0, D_OUT_PAD - D_OUT)))
    b1r = b1.reshape(1, D)
    b2r = jnp.pad(b2, (0, D_OUT_PAD - D_OUT)).reshape(1, D_OUT_PAD)

    degp = _make_deg_kernel()(dst)
    g1, dinv = _m1(x, W1, degp)
    p1 = _make_agg_kernel()(g1, src, dst)
    u = _m2(p1, g1, dinv, b1r)
    p2 = _make_agg_kernel()(u, src, dst)
    out = _e3(p2, u, dinv, b2r, w2p)
    return out[:, :D_OUT]


def kernel(x, edge_index, W1, b1, W2, b2):
    return _run(x, edge_index, W1, b1, W2, b2)


# speedup vs baseline: 27.5885x; 1.8264x over previous
import functools

import jax
import jax.numpy as jnp
from jax import lax
from jax.experimental import pallas as pl
from jax.experimental.pallas import tpu as pltpu
from jax.experimental.pallas import tpu_sc as plsc

N = 10000
E = 320000
D = 128
DEG_W = 16
D_OUT = 40
D_OUT_PAD = 48

NC = 2
NS = 16
NT = NC * NS
CHUNK = 128
BASE_CHUNKS = 78
EXTRA_BASE = NT * BASE_CHUNKS * CHUNK
EXTRA_TILES = (E - EXTRA_BASE) // CHUNK

ROWS_A = 624
TAIL0 = NS * ROWS_A
TAIL = N - TAIL0
Z_LENS = (128, 128, 128, 128, 112)


def _sc_mesh():
    return plsc.VectorSubcoreMesh(
        core_axis_name="c", subcore_axis_name="s", num_cores=NC, num_subcores=NS
    )


def _fill_rows(rows_ref, nrows, width, value):
    vec = jnp.full((16,), value, jnp.float32)

    @pl.loop(0, nrows)
    def _(r):
        @pl.loop(0, width // 16)
        def _(l):
            rows_ref.at[r][pl.ds(l * 16, 16)] = vec


def _zero_acc_slice(rows, acc, sid):
    row0 = sid * ROWS_A
    off = 0
    for ln in Z_LENS:
        pltpu.sync_copy(rows.at[pl.ds(0, ln)], acc.at[pl.ds(row0 + off, ln)])
        off += ln

    @pl.when(sid == 0)
    def _():
        pltpu.sync_copy(rows.at[pl.ds(0, TAIL)], acc.at[pl.ds(TAIL0, TAIL)])


def _write_back(acc, out_hbm, cid, sid):
    row0 = sid * ROWS_A
    pltpu.sync_copy(acc.at[pl.ds(row0, ROWS_A)],
                    out_hbm.at[cid, pl.ds(row0, ROWS_A)])

    @pl.when(sid == 0)
    def _():
        pltpu.sync_copy(acc.at[pl.ds(TAIL0, TAIL)],
                        out_hbm.at[cid, pl.ds(TAIL0, TAIL)])


def _chunk_plan(cid, sid):
    tid = cid * NS + sid
    e0 = tid * (BASE_CHUNKS * CHUNK)
    n = BASE_CHUNKS + jnp.where(tid < EXTRA_TILES, 1, 0)

    def cbase(j):
        return jnp.where(j < BASE_CHUNKS, e0 + j * CHUNK,
                         EXTRA_BASE + tid * CHUNK)

    return n, cbase


@functools.cache
def _make_deg_kernel():
    @functools.partial(
        pl.kernel,
        out_type=jax.ShapeDtypeStruct((NC, N, DEG_W), jnp.float32),
        mesh=_sc_mesh(),
        scratch_types=[
            pltpu.VMEM((CHUNK,), jnp.int32),
            pltpu.VMEM((CHUNK,), jnp.int32),
            pltpu.VMEM((CHUNK, DEG_W), jnp.float32),
            pltpu.VMEM_SHARED((N, DEG_W), jnp.float32),
            pltpu.SemaphoreType.DMA,
            pltpu.SemaphoreType.DMA,
        ],
    )
    def deg_kernel(dst_hbm, out_hbm, idx0, idx1, rows, acc, sem0, sem1):
        cid = lax.axis_index("c")
        sid = lax.axis_index("s")
        n, cbase = _chunk_plan(cid, sid)
        idx = (idx0, idx1)
        sem = (sem0, sem1)

        _fill_rows(rows, CHUNK, DEG_W, 0.0)
        _zero_acc_slice(rows, acc, sid)
        plsc.subcore_barrier()
        _fill_rows(rows, CHUNK, DEG_W, 1.0)

        def idx_start(j, s):
            pltpu.async_copy(dst_hbm.at[pl.ds(cbase(j), CHUNK)], idx[s], sem[s])

        def idx_wait(s):
            pltpu.make_async_copy(dst_hbm.at[pl.ds(0, CHUNK)], idx[s],
                                  sem[s]).wait()

        def scat(s):
            pltpu.sync_copy(rows, acc.at[idx[s]], add=True)

        idx_start(0, 0)
        idx_start(1, 1)

        @pl.loop(0, BASE_CHUNKS // 2)
        def _(k):
            j0 = 2 * k
            idx_wait(0)
            scat(0)

            @pl.when(j0 + 2 < n)
            def _():
                idx_start(j0 + 2, 0)

            idx_wait(1)
            scat(1)

            @pl.when(j0 + 3 < n)
            def _():
                idx_start(j0 + 3, 1)

        @pl.when(BASE_CHUNKS < n)
        def _():
            idx_wait(0)
            scat(0)

        plsc.subcore_barrier()
        _write_back(acc, out_hbm, cid, sid)

    return deg_kernel


@functools.cache
def _make_agg_kernel():
    @functools.partial(
        pl.kernel,
        out_type=jax.ShapeDtypeStruct((NC, N, D), jnp.float32),
        mesh=_sc_mesh(),
        scratch_types=[
            pltpu.VMEM((CHUNK,), jnp.int32),
            pltpu.VMEM((CHUNK,), jnp.int32),
            pltpu.VMEM((CHUNK,), jnp.int32),
            pltpu.VMEM((CHUNK,), jnp.int32),
            pltpu.VMEM((CHUNK, D), jnp.float32),
            pltpu.VMEM((CHUNK, D), jnp.float32),
            pltpu.VMEM_SHARED((N, D), jnp.float32),
            pltpu.SemaphoreType.DMA,
            pltpu.SemaphoreType.DMA,
            pltpu.SemaphoreType.DMA,
            pltpu.SemaphoreType.DMA,
        ],
    )
    def agg_kernel(g_hbm, src_hbm, dst_hbm, out_hbm,
                   idx_s0, idx_s1, idx_d0, idx_d1, rows0, rows1, acc,
                   sem_i0, sem_i1, sem_g0, sem_g1):
        cid = lax.axis_index("c")
        sid = lax.axis_index("s")
        n, cbase = _chunk_plan(cid, sid)
        idx_s = (idx_s0, idx_s1)
        idx_d = (idx_d0, idx_d1)
        rows = (rows0, rows1)
        sem_i = (sem_i0, sem_i1)
        sem_g = (sem_g0, sem_g1)

        _fill_rows(rows0, CHUNK, D, 0.0)
        _zero_acc_slice(rows0, acc, sid)
        plsc.subcore_barrier()

        def idx_start(j, s):
            b = cbase(j)
            pltpu.async_copy(src_hbm.at[pl.ds(b, CHUNK)], idx_s[s], sem_i[s])
            pltpu.async_copy(dst_hbm.at[pl.ds(b, CHUNK)], idx_d[s], sem_i[s])

        def idx_wait(s):
            pltpu.make_async_copy(src_hbm.at[pl.ds(0, CHUNK)], idx_s[s],
                                  sem_i[s]).wait()
            pltpu.make_async_copy(dst_hbm.at[pl.ds(0, CHUNK)], idx_d[s],
                                  sem_i[s]).wait()

        def g_start(s):
            pltpu.async_copy(g_hbm.at[idx_s[s]], rows[s], sem_g[s])

        def g_wait(s):
            pltpu.make_async_copy(g_hbm.at[idx_s[s]], rows[s], sem_g[s]).wait()

        def scat(s):
            pltpu.sync_copy(rows[s], acc.at[idx_d[s]], add=True)

        idx_start(0, 0)
        idx_start(1, 1)
        idx_wait(0)
        g_start(0)

        @pl.loop(0, BASE_CHUNKS // 2)
        def _(k):
            j0 = 2 * k
            g_wait(0)
            idx_wait(1)
            g_start(1)

            @pl.when(j0 + 2 < n)
            def _():
                idx_start(j0 + 2, 0)

            scat(0)
            g_wait(1)

            @pl.when(j0 + 2 < n)
            def _():
                idx_wait(0)
                g_start(0)

            @pl.when(j0 + 3 < n)
            def _():
                idx_start(j0 + 3, 1)

            scat(1)

        @pl.when(BASE_CHUNKS < n)
        def _():
            g_wait(0)
            scat(0)

        plsc.subcore_barrier()
        _write_back(acc, out_hbm, cid, sid)

    return agg_kernel


def _m1_body(x_ref, w_ref, degp_ref, g1_ref, dinv_ref):
    deg = degp_ref[0, :, 0:1] + degp_ref[1, :, 0:1] + 1.0
    dinv = lax.rsqrt(deg)
    dinv_ref[...] = dinv
    h = jnp.dot(x_ref[...], w_ref[...], preferred_element_type=jnp.float32)
    g1_ref[...] = h * dinv


def _m2_body(p_ref, g1_ref, dinv_ref, b1_ref, u_ref):
    s = p_ref[0] + p_ref[1] + g1_ref[...]
    dinv = dinv_ref[...]
    h1 = jnp.maximum(s * dinv + b1_ref[...], 0.0)
    u_ref[...] = h1 * dinv


def _e3_body(p_ref, u_ref, dinv_ref, b2_ref, w2_ref, out_ref):
    s = p_ref[0] + p_ref[1] + u_ref[...]
    h2 = jnp.dot(s, w2_ref[...], preferred_element_type=jnp.float32)
    out_ref[...] = h2 * dinv_ref[...] + b2_ref[...]


_m1 = pl.pallas_call(
    _m1_body,
    out_shape=(jax.ShapeDtypeStruct((N, D), jnp.float32),
               jax.ShapeDtypeStruct((N, 1), jnp.float32)),
)
_m2 = pl.pallas_call(
    _m2_body,
    out_shape=jax.ShapeDtypeStruct((N, D), jnp.float32),
)
_e3 = pl.pallas_call(
    _e3_body,
    out_shape=jax.ShapeDtypeStruct((N, D_OUT_PAD), jnp.float32),
)


@jax.jit
def _run(x, edge_index, W1, b1, W2, b2):
    ei = edge_index.astype(jnp.int32)
    src = ei[0]
    dst = ei[1]
    w2p = jnp.pad(W2, ((0, 0), (0, D_OUT_PAD - D_OUT)))
    b1r = b1.reshape(1, D)
    b2r = jnp.pad(b2, (0, D_OUT_PAD - D_OUT)).reshape(1, D_OUT_PAD)

    degp = _make_deg_kernel()(dst)
    g1, dinv = _m1(x, W1, degp)
    p1 = _make_agg_kernel()(g1, src, dst)
    u = _m2(p1, g1, dinv, b1r)
    p2 = _make_agg_kernel()(u, src, dst)
    out = _e3(p2, u, dinv, b2r, w2p)
    return out[:, :D_OUT]


def kernel(x, edge_index, W1, b1, W2, b2):
    return _run(x, edge_index, W1, b1, W2, b2)
